# Initial kernel scaffold; baseline (speedup 1.0000x reference)
import jax
import jax.numpy as jnp
from jax.experimental import pallas as pl


def kernel(input_ids, embed, norm_w, in_proj_w, conv_w, conv_b, x_proj_w, dt_proj_w, dt_proj_b, A_log, D, out_proj_w, norm_f_w):
    raise NotImplementedError("write your pallas kernel here")


# speedup vs baseline: 11.9987x; 11.9987x over previous
import functools

import jax
import jax.numpy as jnp
from jax.experimental import pallas as pl
from jax.experimental.pallas import tpu as pltpu

_INTERPRET = False

_LOG2E = 1.4426950408889634
_CONTRACT_LAST = (((1,), (1,)), ((), ()))


def _bf(x):
    return x.astype(jnp.bfloat16)


def _gather_body(ids_ref, emb_ref, out_ref, sem):
    n = out_ref.shape[0]

    def issue(i, _):
        idx = ids_ref[i]
        pltpu.make_async_copy(emb_ref.at[pl.ds(idx, 1), :],
                              out_ref.at[pl.ds(i, 1), :], sem).start()
        return 0

    jax.lax.fori_loop(0, n, issue, 0)

    def drain(i, _):
        pltpu.make_async_copy(emb_ref.at[pl.ds(0, 1), :],
                              out_ref.at[pl.ds(0, 1), :], sem).wait()
        return 0

    jax.lax.fori_loop(0, n, drain, 0)


def _embed_gather(ids_flat, embed):
    m = ids_flat.shape[0]
    dm = embed.shape[1]
    return pl.pallas_call(
        _gather_body,
        out_shape=jax.ShapeDtypeStruct((m, dm), jnp.float32),
        in_specs=[pl.BlockSpec(memory_space=pltpu.SMEM),
                  pl.BlockSpec(memory_space=pl.ANY)],
        out_specs=pl.BlockSpec(memory_space=pltpu.VMEM),
        scratch_shapes=[pltpu.SemaphoreType.DMA],
        name="embed_gather",
        interpret=_INTERPRET,
    )(ids_flat, embed)


def _silu(v):
    return v * jax.nn.sigmoid(v)


def _inproj_body(x_ref, nw_ref, wu_ref, wz_ref, cw_ref, cb_ref,
                 u_ref, zs_ref, *, seg):
    xv = x_ref[...]
    ms = jnp.mean(xv * xv, axis=-1, keepdims=True)
    hb = _bf(xv * jax.lax.rsqrt(ms + 1e-5) * nw_ref[...])

    xz_u = jax.lax.dot_general(hb, _bf(wu_ref[...]), _CONTRACT_LAST,
                               preferred_element_type=jnp.float32)
    xz_z = jax.lax.dot_general(hb, _bf(wz_ref[...]), _CONTRACT_LAST,
                               preferred_element_type=jnp.float32)

    m, w = xz_u.shape
    dc = cw_ref.shape[0]
    row = jax.lax.broadcasted_iota(jnp.int32, (m, 1), 0)
    pos = jax.lax.rem(row, seg)
    uc = xz_u * cw_ref[dc - 1:dc, :]
    for s in range(1, dc):
        shifted = jnp.concatenate(
            [jnp.zeros((s, w), jnp.float32), xz_u[:-s, :]], axis=0)
        shifted = jnp.where(pos >= s, shifted, 0.0)
        uc = uc + shifted * cw_ref[dc - 1 - s:dc - s, :]
    u_ref[...] = _silu(uc + cb_ref[...])
    zs_ref[...] = _silu(xz_z)


def _inproj(x, norm_w_l, in_proj_w_l, cw_l, cb_l, *, di, seg, wchunk):
    m, dm = x.shape
    nc = di // wchunk
    grid = (nc,)
    kern = functools.partial(_inproj_body, seg=seg)
    return pl.pallas_call(
        kern,
        grid=grid,
        in_specs=[
            pl.BlockSpec((m, dm), lambda c: (0, 0)),
            pl.BlockSpec((1, dm), lambda c: (0, 0)),
            pl.BlockSpec((wchunk, dm), lambda c: (c, 0)),
            pl.BlockSpec((wchunk, dm), lambda c, _nc=nc: (c + _nc, 0)),
            pl.BlockSpec((cw_l.shape[0], wchunk), lambda c: (0, c)),
            pl.BlockSpec((1, wchunk), lambda c: (0, c)),
        ],
        out_specs=[
            pl.BlockSpec((m, wchunk), lambda c: (0, c)),
            pl.BlockSpec((m, wchunk), lambda c: (0, c)),
        ],
        out_shape=[jax.ShapeDtypeStruct((m, di), jnp.float32),
                   jax.ShapeDtypeStruct((m, di), jnp.float32)],
        compiler_params=pltpu.CompilerParams(
            dimension_semantics=("arbitrary",),
            vmem_limit_bytes=100 * 1024 * 1024,
        ),
        name="rms_inproj_conv",
        interpret=_INTERPRET,
    )(x, norm_w_l, in_proj_w_l, in_proj_w_l, cw_l, cb_l)


def _xdt_body(u_ref, wdtr_ref, wb_ref, wc_ref, wdt_ref, dtb_ref,
              dt_ref, dtu_ref, bm_ref, cm_ref):
    uv = u_ref[...]
    ub = _bf(uv)
    dtr = jax.lax.dot_general(ub, _bf(wdtr_ref[...]), _CONTRACT_LAST,
                              preferred_element_type=jnp.float32)
    bm_ref[...] = jax.lax.dot_general(ub, _bf(wb_ref[...]), _CONTRACT_LAST,
                                      preferred_element_type=jnp.float32)
    cm_ref[...] = jax.lax.dot_general(ub, _bf(wc_ref[...]), _CONTRACT_LAST,
                                      preferred_element_type=jnp.float32)
    dtx = jax.lax.dot_general(_bf(dtr), _bf(wdt_ref[...]), _CONTRACT_LAST,
                              preferred_element_type=jnp.float32)
    dtx = dtx + dtb_ref[...]
    dt = jnp.where(dtx > 20.0, dtx, jnp.log1p(jnp.exp(dtx)))
    dt_ref[...] = dt
    dtu_ref[...] = dt * uv


def _xdt(u, wdtr, wb, wc, wdt, dtb, *, ds):
    m, di = u.shape
    dtrk = wdtr.shape[0]
    return pl.pallas_call(
        _xdt_body,
        out_shape=[jax.ShapeDtypeStruct((m, di), jnp.float32),
                   jax.ShapeDtypeStruct((m, di), jnp.float32),
                   jax.ShapeDtypeStruct((m, ds), jnp.float32),
                   jax.ShapeDtypeStruct((m, ds), jnp.float32)],
        compiler_params=pltpu.CompilerParams(
            vmem_limit_bytes=100 * 1024 * 1024,
        ),
        name="xproj_dtproj",
        interpret=_INTERPRET,
    )(u, wdtr, wb, wc, wdt, dtb)


def _scan_body(dt_ref, dtu_ref, u_ref, zs_ref, alog_ref, d_ref,
               bm_ref, cm_ref, yg_ref, *, seg, nb):
    ds = alog_ref.shape[0]
    w = alog_ref.shape[1]
    a_sc = (-_LOG2E) * jnp.exp(alog_ref[...])
    dvec = d_ref[...]
    nblk = seg // 8

    def batch_block(b, blk, h):
        base = b * nblk + blk
        dt8 = dt_ref[base]
        dtu8 = dtu_ref[base]
        bc8 = bm_ref[base]
        cc8 = cm_ref[base]
        ys = []
        for j in range(8):
            dt_row = dt8[j:j + 1, :]
            a = jnp.exp2(a_sc * dt_row)
            dbu = bc8[:, j:j + 1] * dtu8[j:j + 1, :]
            h = a * h + dbu
            ys.append(jnp.sum(cc8[:, j:j + 1] * h, axis=0, keepdims=True))
        y8 = jnp.concatenate(ys, axis=0)
        yg8 = (y8 + u_ref[base] * dvec) * zs_ref[base]
        yg_ref[base] = yg8
        return h

    def body(blk, carry):
        h0, h1 = carry
        h0 = batch_block(0, blk, h0)
        h1 = batch_block(1, blk, h1)
        return (h0, h1)

    z = jnp.zeros((ds, w), jnp.float32)
    jax.lax.fori_loop(0, nblk, body, (z, z))


def _scan(dt3, dtu3, u3, zs3, alogT, dvec2, bm_t8, cm_t8, *, seg, wchunk):
    mb, eight, di = dt3.shape
    m = mb * eight
    nb = m // seg
    ds = alogT.shape[0]
    nc = di // wchunk
    kern = functools.partial(_scan_body, seg=seg, nb=nb)
    yg3 = pl.pallas_call(
        kern,
        grid=(nc,),
        in_specs=[
            pl.BlockSpec((mb, 8, wchunk), lambda c: (0, 0, c)),
            pl.BlockSpec((mb, 8, wchunk), lambda c: (0, 0, c)),
            pl.BlockSpec((mb, 8, wchunk), lambda c: (0, 0, c)),
            pl.BlockSpec((mb, 8, wchunk), lambda c: (0, 0, c)),
            pl.BlockSpec((ds, wchunk), lambda c: (0, c)),
            pl.BlockSpec((1, wchunk), lambda c: (0, c)),
            pl.BlockSpec((mb, ds, 8), lambda c: (0, 0, 0)),
            pl.BlockSpec((mb, ds, 8), lambda c: (0, 0, 0)),
        ],
        out_specs=pl.BlockSpec((mb, 8, wchunk), lambda c: (0, 0, c)),
        out_shape=jax.ShapeDtypeStruct((mb, 8, di), jnp.float32),
        compiler_params=pltpu.CompilerParams(
            dimension_semantics=("arbitrary",),
            vmem_limit_bytes=100 * 1024 * 1024,
        ),
        name="selective_scan",
        interpret=_INTERPRET,
    )(dt3, dtu3, u3, zs3, alogT, dvec2, bm_t8, cm_t8)
    return yg3


def _outproj_body(yg_ref, w_ref, x_ref, o_ref):
    o_ref[...] = x_ref[...] + jax.lax.dot_general(
        _bf(yg_ref[...]), _bf(w_ref[...]), _CONTRACT_LAST,
        preferred_element_type=jnp.float32)


def _outproj(yg, w_out, x):
    m, dm = x.shape
    return pl.pallas_call(
        _outproj_body,
        out_shape=jax.ShapeDtypeStruct((m, dm), jnp.float32),
        compiler_params=pltpu.CompilerParams(
            vmem_limit_bytes=100 * 1024 * 1024,
        ),
        name="outproj_residual",
        interpret=_INTERPRET,
    )(yg, w_out, x)


def _rmsf_body(x_ref, w_ref, o_ref):
    xv = x_ref[...]
    ms = jnp.mean(xv * xv, axis=-1, keepdims=True)
    o_ref[...] = _bf(xv * jax.lax.rsqrt(ms + 1e-5) * w_ref[...])


def _rms_final(x, w):
    m, dm = x.shape
    return pl.pallas_call(
        _rmsf_body,
        out_shape=jax.ShapeDtypeStruct((m, dm), jnp.bfloat16),
        name="rms_final",
        interpret=_INTERPRET,
    )(x, w.reshape(1, dm))


def _lmhead_body(h_ref, e_ref, o_ref):
    o_ref[...] = jax.lax.dot_general(
        h_ref[...], _bf(e_ref[...]), _CONTRACT_LAST,
        preferred_element_type=jnp.float32)


def _lmhead(hf, embed, *, vtile):
    m, dm = hf.shape
    v = embed.shape[0]
    nv = v // vtile
    return pl.pallas_call(
        _lmhead_body,
        grid=(nv,),
        in_specs=[
            pl.BlockSpec((m, dm), lambda i: (0, 0)),
            pl.BlockSpec((vtile, dm), lambda i: (i, 0)),
        ],
        out_specs=pl.BlockSpec((m, vtile), lambda i: (0, i)),
        out_shape=jax.ShapeDtypeStruct((m, v), jnp.float32),
        compiler_params=pltpu.CompilerParams(
            dimension_semantics=("arbitrary",),
            vmem_limit_bytes=100 * 1024 * 1024,
        ),
        name="lm_head",
        interpret=_INTERPRET,
    )(hf, embed)


def kernel(input_ids, embed, norm_w, in_proj_w, conv_w, conv_b, x_proj_w,
           dt_proj_w, dt_proj_b, A_log, D, out_proj_w, norm_f_w):
    bsz, seg = input_ids.shape
    v, dm = embed.shape
    nl, di, ds = A_log.shape
    dtr = dt_proj_w.shape[2]
    m = bsz * seg
    wchunk = 512

    cw = jnp.swapaxes(conv_w[:, :, 0, :], 1, 2)
    alogT = jnp.swapaxes(A_log, 1, 2)
    wdtr = x_proj_w[:, :dtr, :]
    wb = x_proj_w[:, dtr:dtr + ds, :]
    wc = x_proj_w[:, dtr + ds:, :]

    x = _embed_gather(input_ids.reshape(m), embed)

    for l in range(nl):
        u, zs = _inproj(x, norm_w[l].reshape(1, dm), in_proj_w[l],
                        cw[l], conv_b[l].reshape(1, di),
                        di=di, seg=seg, wchunk=wchunk)
        dt, dtu, bm, cm = _xdt(u, wdtr[l], wb[l], wc[l], dt_proj_w[l],
                               dt_proj_b[l].reshape(1, di), ds=ds)
        dt3 = dt.reshape(m // 8, 8, di)
        dtu3 = dtu.reshape(m // 8, 8, di)
        u3 = u.reshape(m // 8, 8, di)
        zs3 = zs.reshape(m // 8, 8, di)
        bm_t8 = jnp.swapaxes(bm.reshape(m // 8, 8, ds), 1, 2)
        cm_t8 = jnp.swapaxes(cm.reshape(m // 8, 8, ds), 1, 2)
        yg3 = _scan(dt3, dtu3, u3, zs3, alogT[l], D[l].reshape(1, di),
                    bm_t8, cm_t8, seg=seg, wchunk=wchunk)
        x = _outproj(yg3.reshape(m, di), out_proj_w[l], x)

    hf = _rms_final(x, norm_f_w)
    logits = _lmhead(hf, embed, vtile=1280)
    return logits.reshape(bsz, seg, v)
